# baseline (speedup 1.0000x reference)
import functools

import jax
import jax.numpy as jnp
from jax import lax
from jax.experimental import pallas as pl
from jax.experimental.pallas import tpu as pltpu
from jax.experimental.pallas import tpu_sc as plsc

_NC, _NS = 2, 16
_NW = _NC * _NS
_ROWS = 8 * 2048
_C = 512
_RPW = _ROWS // _NW
_R = 64
_NCHUNK = _RPW // _R


def _make_sc_repeat():
    mesh = plsc.VectorSubcoreMesh(core_axis_name="c", subcore_axis_name="s")

    @functools.partial(
        pl.kernel,
        mesh=mesh,
        out_type=jax.ShapeDtypeStruct((_ROWS, 2, _C), jnp.float32),
        scratch_types=[
            pltpu.VMEM((_R, 1, _C), jnp.float32),
            pltpu.VMEM((_R, 1, _C), jnp.float32),
            pltpu.SemaphoreType.DMA,
            pltpu.SemaphoreType.DMA,
            pltpu.SemaphoreType.DMA,
            pltpu.SemaphoreType.DMA,
        ],
    )
    def sc_repeat(x_hbm, o_hbm, buf_a, buf_b, rsem_a, rsem_b, wsem_a, wsem_b):
        w = lax.axis_index("s") * _NC + lax.axis_index("c")
        base = w * _RPW
        bufs = (buf_a, buf_b)
        rsems = (rsem_a, rsem_b)
        wsems = (wsem_a, wsem_b)

        def read(i):
            r0 = base + i * _R
            return pltpu.async_copy(
                x_hbm.at[pl.ds(r0, _R)], bufs[i % 2], rsems[i % 2])

        def write(i, half):
            r0 = base + i * _R
            return pltpu.async_copy(
                bufs[i % 2], o_hbm.at[pl.ds(r0, _R), pl.ds(half, 1)],
                wsems[i % 2])

        pending_w = [None] * _NCHUNK
        rd = read(0)
        for i in range(_NCHUNK):
            rd.wait()
            pending_w[i] = (write(i, 0), write(i, 1))
            if i + 1 < _NCHUNK:
                if i >= 1:
                    for h in pending_w[i - 1]:
                        h.wait()
                rd = read(i + 1)
        for i in (_NCHUNK - 2, _NCHUNK - 1):
            if i >= 0:
                for h in pending_w[i]:
                    h.wait()

    return sc_repeat


_sc_repeat = _make_sc_repeat()


def kernel(x, duration):
    del duration
    B, T, C = x.shape
    x3 = x.reshape(B * T, 1, C)
    out = _sc_repeat(x3)
    return out.reshape(B, 2 * T, C)


# speedup vs baseline: 1.7236x; 1.7236x over previous
import functools

import jax
import jax.numpy as jnp
from jax import lax
from jax.experimental import pallas as pl
from jax.experimental.pallas import tpu as pltpu
from jax.experimental.pallas import tpu_sc as plsc

_NC, _NS = 2, 16
_NW = _NC * _NS
_B, _T, _C = 8, 2048, 512
_WPB = _NW // _B
_RPW = _T // _WPB
_R = 64
_NCHUNK = _RPW // _R


def _make_sc_repeat():
    mesh = plsc.VectorSubcoreMesh(core_axis_name="c", subcore_axis_name="s")

    @functools.partial(
        pl.kernel,
        mesh=mesh,
        out_type=jax.ShapeDtypeStruct((_B, _T, 2, _C), jnp.float32),
        scratch_types=[
            pltpu.VMEM((_R, _C), jnp.float32),
            pltpu.VMEM((_R, _C), jnp.float32),
            pltpu.SemaphoreType.DMA,
            pltpu.SemaphoreType.DMA,
            pltpu.SemaphoreType.DMA,
            pltpu.SemaphoreType.DMA,
        ],
    )
    def sc_repeat(x_hbm, o_hbm, buf_a, buf_b, rsem_a, rsem_b, wsem_a, wsem_b):
        w = lax.axis_index("s") * _NC + lax.axis_index("c")
        b = w // _WPB
        t0 = (w % _WPB) * _RPW
        bufs = (buf_a, buf_b)
        rsems = (rsem_a, rsem_b)
        wsems = (wsem_a, wsem_b)

        def read(i):
            return pltpu.async_copy(
                x_hbm.at[b, pl.ds(t0 + i * _R, _R)], bufs[i % 2], rsems[i % 2])

        def write(i, half):
            return pltpu.async_copy(
                bufs[i % 2], o_hbm.at[b, pl.ds(t0 + i * _R, _R), half],
                wsems[i % 2])

        pending_w = [None] * _NCHUNK
        rd = read(0)
        for i in range(_NCHUNK):
            rd.wait()
            pending_w[i] = (write(i, 0), write(i, 1))
            if i + 1 < _NCHUNK:
                if i >= 1:
                    for h in pending_w[i - 1]:
                        h.wait()
                rd = read(i + 1)
        for i in (_NCHUNK - 2, _NCHUNK - 1):
            if i >= 0:
                for h in pending_w[i]:
                    h.wait()

    return sc_repeat


_sc_repeat = _make_sc_repeat()


def kernel(x, duration):
    del duration
    B, T, C = x.shape
    out = _sc_repeat(x)
    return out.reshape(B, 2 * T, C)


# speedup vs baseline: 4.1860x; 2.4286x over previous
import functools

import jax
import jax.numpy as jnp
from jax import lax
from jax.experimental import pallas as pl
from jax.experimental.pallas import tpu as pltpu
from jax.experimental.pallas import tpu_sc as plsc

_NC, _NS = 2, 16
_NW = _NC * _NS
_B, _T, _C = 8, 2048, 512
_ROWS = _B * _T
_RPW = _ROWS // _NW
_R = 64
_NCHUNK = _RPW // _R
_L = 16


def _make_sc_repeat():
    mesh = plsc.VectorSubcoreMesh(core_axis_name="c", subcore_axis_name="s")

    @functools.partial(
        pl.kernel,
        mesh=mesh,
        out_type=jax.ShapeDtypeStruct((2 * _ROWS, _C), jnp.float32),
        scratch_types=[
            pltpu.VMEM((_R, _C), jnp.float32),
            pltpu.VMEM((_R, _C), jnp.float32),
            pltpu.VMEM((_R,), jnp.int32),
            pltpu.VMEM((_R,), jnp.int32),
            pltpu.VMEM((_R,), jnp.int32),
            pltpu.VMEM((_R,), jnp.int32),
            pltpu.SemaphoreType.DMA,
            pltpu.SemaphoreType.DMA,
            pltpu.SemaphoreType.DMA,
            pltpu.SemaphoreType.DMA,
        ],
    )
    def sc_repeat(x_hbm, o_hbm, buf_a, buf_b, ie_a, ie_b, io_a, io_b,
                  rsem_a, rsem_b, wsem_a, wsem_b):
        w = lax.axis_index("s") * _NC + lax.axis_index("c")
        base = w * _RPW
        bufs = (buf_a, buf_b)
        idx_e = (ie_a, ie_b)
        idx_o = (io_a, io_b)
        rsems = (rsem_a, rsem_b)
        wsems = (wsem_a, wsem_b)

        def fill_idx(i):
            r0 = base + i * _R
            for k in range(_R // _L):
                v = 2 * (lax.iota(jnp.int32, _L) + (r0 + k * _L))
                idx_e[i % 2][pl.ds(k * _L, _L)] = v
                idx_o[i % 2][pl.ds(k * _L, _L)] = v + 1

        def read(i):
            r0 = base + i * _R
            return pltpu.async_copy(
                x_hbm.at[pl.ds(r0, _R)], bufs[i % 2], rsems[i % 2])

        def scatter(i):
            return (
                pltpu.async_copy(bufs[i % 2], o_hbm.at[idx_e[i % 2]],
                                 wsems[i % 2]),
                pltpu.async_copy(bufs[i % 2], o_hbm.at[idx_o[i % 2]],
                                 wsems[i % 2]),
            )

        pending_w = [None] * _NCHUNK
        fill_idx(0)
        rd = read(0)
        for i in range(_NCHUNK):
            rd.wait()
            pending_w[i] = scatter(i)
            if i + 1 < _NCHUNK:
                if i >= 1:
                    for h in pending_w[i - 1]:
                        h.wait()
                fill_idx(i + 1)
                rd = read(i + 1)
        for i in (_NCHUNK - 2, _NCHUNK - 1):
            if i >= 0:
                for h in pending_w[i]:
                    h.wait()

    return sc_repeat


_sc_repeat = _make_sc_repeat()


def kernel(x, duration):
    del duration
    B, T, C = x.shape
    out = _sc_repeat(x.reshape(B * T, C))
    return out.reshape(B, 2 * T, C)


# speedup vs baseline: 4.3956x; 1.0501x over previous
import functools

import jax
import jax.numpy as jnp
from jax import lax
from jax.experimental import pallas as pl
from jax.experimental.pallas import tpu as pltpu
from jax.experimental.pallas import tpu_sc as plsc

_NC, _NS = 2, 16
_NW = _NC * _NS
_B, _T, _C = 8, 2048, 512
_ROWS = _B * _T
_RPW = _ROWS // _NW
_R = 64
_NCHUNK = _RPW // _R
_NBUF = 3
_L = 16


def _make_sc_repeat():
    mesh = plsc.VectorSubcoreMesh(core_axis_name="c", subcore_axis_name="s")

    @functools.partial(
        pl.kernel,
        mesh=mesh,
        out_type=jax.ShapeDtypeStruct((2 * _ROWS, _C), jnp.float32),
        scratch_types=(
            [pltpu.VMEM((_R, _C), jnp.float32)] * _NBUF
            + [pltpu.VMEM((_R,), jnp.int32)] * (2 * _NBUF)
            + [pltpu.SemaphoreType.DMA] * (2 * _NBUF)
        ),
    )
    def sc_repeat(x_hbm, o_hbm, *scratch):
        bufs = scratch[:_NBUF]
        idx_e = scratch[_NBUF:2 * _NBUF]
        idx_o = scratch[2 * _NBUF:3 * _NBUF]
        rsems = scratch[3 * _NBUF:4 * _NBUF]
        wsems = scratch[4 * _NBUF:5 * _NBUF]
        w = lax.axis_index("s") * _NC + lax.axis_index("c")
        base = w * _RPW

        def fill_idx(i):
            r0 = base + i * _R
            for k in range(_R // _L):
                v = 2 * (lax.iota(jnp.int32, _L) + (r0 + k * _L))
                idx_e[i % _NBUF][pl.ds(k * _L, _L)] = v
                idx_o[i % _NBUF][pl.ds(k * _L, _L)] = v + 1

        def read(i):
            r0 = base + i * _R
            return pltpu.async_copy(
                x_hbm.at[pl.ds(r0, _R)], bufs[i % _NBUF], rsems[i % _NBUF])

        def scatter(i):
            return (
                pltpu.async_copy(bufs[i % _NBUF], o_hbm.at[idx_e[i % _NBUF]],
                                 wsems[i % _NBUF]),
                pltpu.async_copy(bufs[i % _NBUF], o_hbm.at[idx_o[i % _NBUF]],
                                 wsems[i % _NBUF]),
            )

        pending_w = [None] * _NCHUNK
        rds = [None] * _NCHUNK
        for i in range(min(_NBUF - 1, _NCHUNK)):
            fill_idx(i)
            rds[i] = read(i)
        for i in range(_NCHUNK):
            rds[i].wait()
            pending_w[i] = scatter(i)
            nxt = i + _NBUF - 1
            if nxt < _NCHUNK:
                if i >= 1:
                    for h in pending_w[i - 1]:
                        h.wait()
                fill_idx(nxt)
                rds[nxt] = read(nxt)
        for i in range(max(0, _NCHUNK - _NBUF), _NCHUNK):
            for h in pending_w[i]:
                h.wait()

    return sc_repeat


_sc_repeat = _make_sc_repeat()


def kernel(x, duration):
    del duration
    B, T, C = x.shape
    out = _sc_repeat(x.reshape(B * T, C))
    return out.reshape(B, 2 * T, C)


# speedup vs baseline: 4.3957x; 1.0000x over previous
import functools

import jax
import jax.numpy as jnp
from jax import lax
from jax.experimental import pallas as pl
from jax.experimental.pallas import tpu as pltpu
from jax.experimental.pallas import tpu_sc as plsc

_NC, _NS = 2, 16
_NW = _NC * _NS
_B, _T, _C = 8, 2048, 512
_ROWS = _B * _T
_RPW = _ROWS // _NW
_R = 32
_NCHUNK = _RPW // _R
_NBUF = 7
_L = 16


def _make_sc_repeat():
    mesh = plsc.VectorSubcoreMesh(core_axis_name="c", subcore_axis_name="s")

    @functools.partial(
        pl.kernel,
        mesh=mesh,
        out_type=jax.ShapeDtypeStruct((2 * _ROWS, _C), jnp.float32),
        scratch_types=(
            [pltpu.VMEM((_R, _C), jnp.float32)] * _NBUF
            + [pltpu.VMEM((_R,), jnp.int32)] * (2 * _NBUF)
            + [pltpu.SemaphoreType.DMA] * (2 * _NBUF)
        ),
    )
    def sc_repeat(x_hbm, o_hbm, *scratch):
        bufs = scratch[:_NBUF]
        idx_e = scratch[_NBUF:2 * _NBUF]
        idx_o = scratch[2 * _NBUF:3 * _NBUF]
        rsems = scratch[3 * _NBUF:4 * _NBUF]
        wsems = scratch[4 * _NBUF:5 * _NBUF]
        w = lax.axis_index("s") * _NC + lax.axis_index("c")
        base = w * _RPW

        def fill_idx(i):
            r0 = base + i * _R
            for k in range(_R // _L):
                v = 2 * (lax.iota(jnp.int32, _L) + (r0 + k * _L))
                idx_e[i % _NBUF][pl.ds(k * _L, _L)] = v
                idx_o[i % _NBUF][pl.ds(k * _L, _L)] = v + 1

        def read(i):
            r0 = base + i * _R
            return pltpu.async_copy(
                x_hbm.at[pl.ds(r0, _R)], bufs[i % _NBUF], rsems[i % _NBUF])

        def scatter(i):
            return (
                pltpu.async_copy(bufs[i % _NBUF], o_hbm.at[idx_e[i % _NBUF]],
                                 wsems[i % _NBUF]),
                pltpu.async_copy(bufs[i % _NBUF], o_hbm.at[idx_o[i % _NBUF]],
                                 wsems[i % _NBUF]),
            )

        pending_w = [None] * _NCHUNK
        rds = [None] * _NCHUNK
        for i in range(min(_NBUF - 1, _NCHUNK)):
            fill_idx(i)
            rds[i] = read(i)
        for i in range(_NCHUNK):
            rds[i].wait()
            pending_w[i] = scatter(i)
            nxt = i + _NBUF - 1
            if nxt < _NCHUNK:
                if i >= 1:
                    for h in pending_w[i - 1]:
                        h.wait()
                fill_idx(nxt)
                rds[nxt] = read(nxt)
        for i in range(max(0, _NCHUNK - _NBUF), _NCHUNK):
            for h in pending_w[i]:
                h.wait()

    return sc_repeat


_sc_repeat = _make_sc_repeat()


def kernel(x, duration):
    del duration
    B, T, C = x.shape
    out = _sc_repeat(x.reshape(B * T, C))
    return out.reshape(B, 2 * T, C)
